# baseline (speedup 1.0000x reference)
import functools

import jax
import jax.numpy as jnp
from jax import lax
from jax.experimental import pallas as pl
from jax.experimental.pallas import tpu as pltpu
from jax.experimental.pallas import tpu_sc as plsc

_N = 10000
_E = 320000
_D = 128
_F = 128
_R = 16

_EH = _E // 2
_NC = 2
_NS = 16
_NW = _NC * _NS
_EPW = _EH // _NW
_CHUNK = 56
_CPW = _EPW // _CHUNK
_TAIL = _EPW - _CPW * _CHUNK
_NT = (_CPW - 2) // 3
_NP = 10240
_RPT = _NP // _NS

_LOG2 = 0.6931471805599453
_LOG2E = 1.4426950408889634


def _ssp(v):
    return jax.nn.softplus(v) - _LOG2


def _ssp_fast(u):
    t = jnp.log2(1.0 + jnp.exp2(u * _LOG2E)) * _LOG2 - _LOG2
    return jnp.where(u > 60.0, u - _LOG2, t)


def _h_body(x_ref, w_ref, o_ref):
    o_ref[...] = lax.dot_general(
        x_ref[...], w_ref[...], (((1,), (1,)), ((), ())),
        preferred_element_type=jnp.float32)


def _compute_h(x, W_in):
    return pl.pallas_call(
        _h_body,
        out_shape=jax.ShapeDtypeStruct((_N, _D), jnp.float32),
    )(x, W_in)


_BE = 4000


def _wc_body(f_ref, c_ref, w1_ref, b1_ref, w2_ref, b2_ref, o_ref):
    t = lax.dot_general(f_ref[:, 0, :], w1_ref[...], (((1,), (1,)), ((), ())),
                        preferred_element_type=jnp.float32)
    t = _ssp_fast(t + b1_ref[...])
    w = lax.dot_general(t.astype(jnp.bfloat16),
                        w2_ref[...].astype(jnp.bfloat16),
                        (((1,), (1,)), ((), ())),
                        preferred_element_type=jnp.float32)
    o_ref[...] = (w + b2_ref[...]) * c_ref[...]


def _make_wc(off_blocks):
    return pl.pallas_call(
        _wc_body,
        grid=(_EH // _BE,),
        in_specs=[
            pl.BlockSpec((_BE, 1, _R), lambda i: (i + off_blocks, 0, 0)),
            pl.BlockSpec((_BE, 1), lambda i: (i + off_blocks, 0)),
            pl.BlockSpec((_F, _R), lambda i: (0, 0)),
            pl.BlockSpec((1, _F), lambda i: (0, 0)),
            pl.BlockSpec((_F, _F), lambda i: (0, 0)),
            pl.BlockSpec((1, _F), lambda i: (0, 0)),
        ],
        out_specs=pl.BlockSpec((_BE, _F), lambda i: (i, 0)),
        out_shape=jax.ShapeDtypeStruct((_EH, _F), jnp.float32),
    )


_compute_wc_a = _make_wc(0)
_compute_wc_b = _make_wc(_EH // _BE)


def _make_sc_aggregate(eoff):
    mesh = plsc.VectorSubcoreMesh(core_axis_name="c", subcore_axis_name="s")

    @functools.partial(
        pl.kernel,
        out_type=jax.ShapeDtypeStruct((_NC * _NP, _D), jnp.float32),
        mesh=mesh,
        scratch_types=[
            pltpu.VMEM((3, _CHUNK), jnp.int32),
            pltpu.VMEM((3, _CHUNK), jnp.int32),
            pltpu.VMEM((_TAIL,), jnp.int32),
            pltpu.VMEM((_TAIL,), jnp.int32),
            pltpu.VMEM((_CHUNK, _D), jnp.float32),
            pltpu.VMEM((_CHUNK, _D), jnp.float32),
            pltpu.VMEM((_CHUNK, _D), jnp.float32),
            pltpu.VMEM((_CHUNK, _D), jnp.float32),
            pltpu.VMEM((_CHUNK, _D), jnp.float32),
            pltpu.VMEM((_CHUNK, _D), jnp.float32),
            pltpu.VMEM_SHARED((_NP, _D), jnp.float32),
            pltpu.SemaphoreType.DMA,
            pltpu.SemaphoreType.DMA,
            pltpu.SemaphoreType.DMA,
            pltpu.SemaphoreType.DMA,
            pltpu.SemaphoreType.DMA,
            pltpu.SemaphoreType.DMA,
            pltpu.SemaphoreType.DMA,
            pltpu.SemaphoreType.DMA,
            pltpu.SemaphoreType.DMA,
            pltpu.SemaphoreType.DMA,
        ],
    )
    def k(h_hbm, wc_hbm, ii_hbm, ij_hbm, z_hbm, out_hbm,
          ii_v, ij_v, ii_t, ij_t, r0, r1, r2, w0, w1, w2, acc_sh,
          sem_z, si0, si1, si2, sm0, sm1, sm2, ss0, ss1, ss2):
        cid = lax.axis_index("c")
        sid = lax.axis_index("s")
        wid = sid * _NC + cid
        rbufs = (r0, r1, r2)
        wbufs = (w0, w1, w2)
        isems = (si0, si1, si2)
        msems = (sm0, sm1, sm2)
        ssems = (ss0, ss1, ss2)

        pltpu.async_copy(z_hbm, acc_sh.at[pl.ds(sid * _RPT, _RPT)], sem_z)

        def lbase(c):
            return wid * _EPW + c * _CHUNK

        def issue_idx(c, b):
            pltpu.async_copy(ii_hbm.at[pl.ds(eoff + lbase(c), _CHUNK)],
                             ii_v.at[b], isems[b])
            pltpu.async_copy(ij_hbm.at[pl.ds(eoff + lbase(c), _CHUNK)],
                             ij_v.at[b], isems[b])

        def wait_idx(c, b):
            pltpu.make_async_copy(ii_hbm.at[pl.ds(eoff + lbase(c), _CHUNK)],
                                  ii_v.at[b], isems[b]).wait()
            pltpu.make_async_copy(ij_hbm.at[pl.ds(eoff + lbase(c), _CHUNK)],
                                  ij_v.at[b], isems[b]).wait()

        def issue_main(c, b):
            pltpu.async_copy(h_hbm.at[ij_v.at[b]], rbufs[b], msems[b])
            pltpu.async_copy(wc_hbm.at[pl.ds(lbase(c), _CHUNK)],
                             wbufs[b], msems[b])

        def wait_main(c, b):
            pltpu.make_async_copy(h_hbm.at[ij_v.at[b]], rbufs[b],
                                  msems[b]).wait()
            pltpu.make_async_copy(wc_hbm.at[pl.ds(lbase(c), _CHUNK)],
                                  wbufs[b], msems[b]).wait()

        def issue_scatter(c, b):
            pltpu.async_copy(rbufs[b], acc_sh.at[ii_v.at[b]], ssems[b],
                             add=True)

        def wait_scatter(c, b):
            pltpu.make_async_copy(rbufs[b], acc_sh.at[ii_v.at[b]],
                                  ssems[b]).wait()

        def compute(b, n=_CHUNK):
            rb, wb = rbufs[b], wbufs[b]

            @pl.loop(0, n)
            def _(e):
                for j in range(0, _D, 16):
                    slc = (pl.ds(e, 1), pl.ds(j, 16))
                    rb.at[slc][...] = rb.at[slc][...] * wb.at[slc][...]

        pltpu.make_async_copy(z_hbm, acc_sh.at[pl.ds(sid * _RPT, _RPT)],
                              sem_z).wait()
        plsc.subcore_barrier()

        issue_idx(0, 0)
        issue_idx(1, 1)
        wait_idx(0, 0)
        issue_main(0, 0)

        @pl.loop(0, _NT)
        def _(t):
            for j in range(3):
                c = t * 3 + j
                b = j
                bn = (j + 2) % 3

                wait_idx(c + 1, (b + 1) % 3)
                issue_main(c + 1, (b + 1) % 3)

                wait_main(c, b)
                compute(b)

                if j == 0:
                    @pl.when(t > 0)
                    def _():
                        wait_scatter(c - 1, bn)
                        issue_idx(c + 2, bn)

                    @pl.when(t == 0)
                    def _():
                        issue_idx(c + 2, bn)
                else:
                    wait_scatter(c - 1, bn)
                    if j == 2:
                        @pl.when(t < _NT - 1)
                        def _():
                            issue_idx(c + 2, bn)
                    else:
                        issue_idx(c + 2, bn)

                issue_scatter(c, b)

        ca = 3 * _NT
        cb = ca + 1
        issue_idx(cb, 1)
        wait_idx(cb, 1)
        issue_main(cb, 1)
        wait_main(ca, 0)
        compute(0)
        wait_scatter(ca - 1, 2)
        issue_scatter(ca, 0)

        tbase = eoff + wid * _EPW + _CPW * _CHUNK
        pltpu.async_copy(ii_hbm.at[pl.ds(tbase, _TAIL)], ii_t, si2)
        pltpu.async_copy(ij_hbm.at[pl.ds(tbase, _TAIL)], ij_t, si2)
        wait_main(cb, 1)
        compute(1)
        wait_scatter(ca, 0)
        issue_scatter(cb, 1)

        pltpu.make_async_copy(ii_hbm.at[pl.ds(tbase, _TAIL)], ii_t,
                              si2).wait()
        pltpu.make_async_copy(ij_hbm.at[pl.ds(tbase, _TAIL)], ij_t,
                              si2).wait()
        pltpu.async_copy(h_hbm.at[ij_t], r2.at[pl.ds(0, _TAIL)], sm2)
        pltpu.async_copy(wc_hbm.at[pl.ds(wid * _EPW + _CPW * _CHUNK, _TAIL)],
                         w2.at[pl.ds(0, _TAIL)], sm2)
        pltpu.make_async_copy(h_hbm.at[ij_t], r2.at[pl.ds(0, _TAIL)],
                              sm2).wait()
        pltpu.make_async_copy(
            wc_hbm.at[pl.ds(wid * _EPW + _CPW * _CHUNK, _TAIL)],
            w2.at[pl.ds(0, _TAIL)], sm2).wait()
        compute(2, n=_TAIL)
        wait_scatter(cb, 1)
        pltpu.async_copy(r2.at[pl.ds(0, _TAIL)], acc_sh.at[ii_t], ss2,
                         add=True)
        pltpu.make_async_copy(r2.at[pl.ds(0, _TAIL)], acc_sh.at[ii_t],
                              ss2).wait()

        plsc.subcore_barrier()
        pltpu.sync_copy(acc_sh.at[pl.ds(sid * _RPT, _RPT)],
                        out_hbm.at[pl.ds(cid * _NP + sid * _RPT, _RPT)])

    return k


_sc_aggregate_a = _make_sc_aggregate(0)
_sc_aggregate_b = _make_sc_aggregate(_EH)


def _out_body(pa_ref, pb_ref, w1_ref, b1_ref, w2_ref, b2_ref, o_ref):
    agg = (pa_ref[0, :_N, :] + pa_ref[1, :_N, :]
           + pb_ref[0, :_N, :] + pb_ref[1, :_N, :])
    t = lax.dot_general(agg, w1_ref[...], (((1,), (1,)), ((), ())),
                        preferred_element_type=jnp.float32)
    t = _ssp(t + b1_ref[...])
    o = lax.dot_general(t, w2_ref[...], (((1,), (1,)), ((), ())),
                        preferred_element_type=jnp.float32)
    o_ref[...] = o + b2_ref[...]


def _out_mlp(pa, pb, Wo1, bo1, Wo2, bo2):
    return pl.pallas_call(
        _out_body,
        out_shape=jax.ShapeDtypeStruct((_N, _D), jnp.float32),
    )(pa, pb, Wo1, bo1, Wo2, bo2)


def kernel(x, pairlist, f_ij, f_ij_cutoff,
           W_in, Wf1, bf1, Wf2, bf2, Wo1, bo1, Wo2, bo2):
    h = _compute_h(x, W_in)
    b1 = bf1.reshape(1, _F)
    b2 = bf2.reshape(1, _F)
    wc_a = _compute_wc_a(f_ij, f_ij_cutoff, Wf1, b1, Wf2, b2)
    wc_b = _compute_wc_b(f_ij, f_ij_cutoff, Wf1, b1, Wf2, b2)
    zeros = jnp.zeros((_RPT, _D), jnp.float32)
    ii = pairlist[0]
    ij = pairlist[1]
    pa = _sc_aggregate_a(h, wc_a, ii, ij, zeros)
    pb = _sc_aggregate_b(h, wc_b, ii, ij, zeros)
    out = _out_mlp(pa.reshape(_NC, _NP, _D), pb.reshape(_NC, _NP, _D),
                   Wo1, bo1.reshape(1, _D), Wo2, bo2.reshape(1, _D))
    return out


# speedup vs baseline: 1.4254x; 1.4254x over previous
import functools

import jax
import jax.numpy as jnp
from jax import lax
from jax.experimental import pallas as pl
from jax.experimental.pallas import tpu as pltpu
from jax.experimental.pallas import tpu_sc as plsc

_N = 10000
_E = 320000
_D = 128
_F = 128
_R = 16

_EH = _E // 2
_NC = 2
_NS = 16
_NW = _NC * _NS
_EPW = _EH // _NW
_CHUNK = 56
_CPW = _EPW // _CHUNK
_TAIL = _EPW - _CPW * _CHUNK
_NT = (_CPW - 2) // 3
_NP = 10240
_RPT = _NP // _NS

_LOG2 = 0.6931471805599453
_LOG2E = 1.4426950408889634


def _ssp(v):
    return jax.nn.softplus(v) - _LOG2


def _ssp_fast(u):
    t = jnp.log2(1.0 + jnp.exp2(u * _LOG2E)) * _LOG2 - _LOG2
    return jnp.where(u > 60.0, u - _LOG2, t)


def _h_body(x_ref, w_ref, o_ref):
    o_ref[...] = lax.dot_general(
        x_ref[...], w_ref[...], (((1,), (1,)), ((), ())),
        preferred_element_type=jnp.float32)


def _compute_h(x, W_in):
    return pl.pallas_call(
        _h_body,
        out_shape=jax.ShapeDtypeStruct((_N, _D), jnp.float32),
    )(x, W_in)


_BE = 4000


def _wc_body(f_ref, c_ref, w1_ref, b1_ref, w2_ref, b2_ref, o_ref):
    t = lax.dot_general(f_ref[...], w1_ref[...], (((1,), (1,)), ((), ())),
                        preferred_element_type=jnp.float32)
    t = _ssp_fast(t + b1_ref[...])
    w = lax.dot_general(t.astype(jnp.bfloat16),
                        w2_ref[...].astype(jnp.bfloat16),
                        (((1,), (1,)), ((), ())),
                        preferred_element_type=jnp.float32)
    o_ref[...] = (w + b2_ref[...]) * c_ref[...]


def _make_wc(off_blocks):
    return pl.pallas_call(
        _wc_body,
        grid=(_EH // _BE,),
        in_specs=[
            pl.BlockSpec((_BE, _R), lambda i: (i, 0)),
            pl.BlockSpec((_BE, 1), lambda i: (i + off_blocks, 0)),
            pl.BlockSpec((_F, _R), lambda i: (0, 0)),
            pl.BlockSpec((1, _F), lambda i: (0, 0)),
            pl.BlockSpec((_F, _F), lambda i: (0, 0)),
            pl.BlockSpec((1, _F), lambda i: (0, 0)),
        ],
        out_specs=pl.BlockSpec((_BE, _F), lambda i: (i, 0)),
        out_shape=jax.ShapeDtypeStruct((_EH, _F), jnp.float32),
    )


_compute_wc_a = _make_wc(0)
_compute_wc_b = _make_wc(_EH // _BE)


def _make_sc_aggregate(eoff):
    mesh = plsc.VectorSubcoreMesh(core_axis_name="c", subcore_axis_name="s")

    @functools.partial(
        pl.kernel,
        out_type=jax.ShapeDtypeStruct((_NC * _NP, _D), jnp.float32),
        mesh=mesh,
        scratch_types=[
            pltpu.VMEM((3, _CHUNK), jnp.int32),
            pltpu.VMEM((3, _CHUNK), jnp.int32),
            pltpu.VMEM((_TAIL,), jnp.int32),
            pltpu.VMEM((_TAIL,), jnp.int32),
            pltpu.VMEM((_CHUNK, _D), jnp.float32),
            pltpu.VMEM((_CHUNK, _D), jnp.float32),
            pltpu.VMEM((_CHUNK, _D), jnp.float32),
            pltpu.VMEM((_CHUNK, _D), jnp.float32),
            pltpu.VMEM((_CHUNK, _D), jnp.float32),
            pltpu.VMEM((_CHUNK, _D), jnp.float32),
            pltpu.VMEM_SHARED((_NP, _D), jnp.float32),
            pltpu.SemaphoreType.DMA,
            pltpu.SemaphoreType.DMA,
            pltpu.SemaphoreType.DMA,
            pltpu.SemaphoreType.DMA,
            pltpu.SemaphoreType.DMA,
            pltpu.SemaphoreType.DMA,
            pltpu.SemaphoreType.DMA,
            pltpu.SemaphoreType.DMA,
            pltpu.SemaphoreType.DMA,
            pltpu.SemaphoreType.DMA,
        ],
    )
    def k(h_hbm, wc_hbm, ii_hbm, ij_hbm, z_hbm, out_hbm,
          ii_v, ij_v, ii_t, ij_t, r0, r1, r2, w0, w1, w2, acc_sh,
          sem_z, si0, si1, si2, sm0, sm1, sm2, ss0, ss1, ss2):
        cid = lax.axis_index("c")
        sid = lax.axis_index("s")
        wid = sid * _NC + cid
        rbufs = (r0, r1, r2)
        wbufs = (w0, w1, w2)
        isems = (si0, si1, si2)
        msems = (sm0, sm1, sm2)
        ssems = (ss0, ss1, ss2)

        pltpu.async_copy(z_hbm, acc_sh.at[pl.ds(sid * _RPT, _RPT)], sem_z)

        def lbase(c):
            return wid * _EPW + c * _CHUNK

        def issue_idx(c, b):
            pltpu.async_copy(ii_hbm.at[pl.ds(eoff + lbase(c), _CHUNK)],
                             ii_v.at[b], isems[b])
            pltpu.async_copy(ij_hbm.at[pl.ds(eoff + lbase(c), _CHUNK)],
                             ij_v.at[b], isems[b])

        def wait_idx(c, b):
            pltpu.make_async_copy(ii_hbm.at[pl.ds(eoff + lbase(c), _CHUNK)],
                                  ii_v.at[b], isems[b]).wait()
            pltpu.make_async_copy(ij_hbm.at[pl.ds(eoff + lbase(c), _CHUNK)],
                                  ij_v.at[b], isems[b]).wait()

        def issue_main(c, b):
            pltpu.async_copy(h_hbm.at[ij_v.at[b]], rbufs[b], msems[b])
            pltpu.async_copy(wc_hbm.at[pl.ds(lbase(c), _CHUNK)],
                             wbufs[b], msems[b])

        def wait_main(c, b):
            pltpu.make_async_copy(h_hbm.at[ij_v.at[b]], rbufs[b],
                                  msems[b]).wait()
            pltpu.make_async_copy(wc_hbm.at[pl.ds(lbase(c), _CHUNK)],
                                  wbufs[b], msems[b]).wait()

        def issue_scatter(c, b):
            pltpu.async_copy(rbufs[b], acc_sh.at[ii_v.at[b]], ssems[b],
                             add=True)

        def wait_scatter(c, b):
            pltpu.make_async_copy(rbufs[b], acc_sh.at[ii_v.at[b]],
                                  ssems[b]).wait()

        def compute(b, n=_CHUNK):
            rb, wb = rbufs[b], wbufs[b]

            @pl.loop(0, n)
            def _(e):
                for j in range(0, _D, 16):
                    slc = (pl.ds(e, 1), pl.ds(j, 16))
                    rb.at[slc][...] = rb.at[slc][...] * wb.at[slc][...]

        pltpu.make_async_copy(z_hbm, acc_sh.at[pl.ds(sid * _RPT, _RPT)],
                              sem_z).wait()
        plsc.subcore_barrier()

        issue_idx(0, 0)
        issue_idx(1, 1)
        wait_idx(0, 0)
        issue_main(0, 0)

        @pl.loop(0, _NT)
        def _(t):
            for j in range(3):
                c = t * 3 + j
                b = j
                bn = (j + 2) % 3

                wait_idx(c + 1, (b + 1) % 3)
                issue_main(c + 1, (b + 1) % 3)

                wait_main(c, b)
                compute(b)

                if j == 0:
                    @pl.when(t > 0)
                    def _():
                        wait_scatter(c - 1, bn)
                        issue_idx(c + 2, bn)

                    @pl.when(t == 0)
                    def _():
                        issue_idx(c + 2, bn)
                else:
                    wait_scatter(c - 1, bn)
                    if j == 2:
                        @pl.when(t < _NT - 1)
                        def _():
                            issue_idx(c + 2, bn)
                    else:
                        issue_idx(c + 2, bn)

                issue_scatter(c, b)

        ca = 3 * _NT
        cb = ca + 1
        issue_idx(cb, 1)
        wait_idx(cb, 1)
        issue_main(cb, 1)
        wait_main(ca, 0)
        compute(0)
        wait_scatter(ca - 1, 2)
        issue_scatter(ca, 0)

        tbase = eoff + wid * _EPW + _CPW * _CHUNK
        pltpu.async_copy(ii_hbm.at[pl.ds(tbase, _TAIL)], ii_t, si2)
        pltpu.async_copy(ij_hbm.at[pl.ds(tbase, _TAIL)], ij_t, si2)
        wait_main(cb, 1)
        compute(1)
        wait_scatter(ca, 0)
        issue_scatter(cb, 1)

        pltpu.make_async_copy(ii_hbm.at[pl.ds(tbase, _TAIL)], ii_t,
                              si2).wait()
        pltpu.make_async_copy(ij_hbm.at[pl.ds(tbase, _TAIL)], ij_t,
                              si2).wait()
        pltpu.async_copy(h_hbm.at[ij_t], r2.at[pl.ds(0, _TAIL)], sm2)
        pltpu.async_copy(wc_hbm.at[pl.ds(wid * _EPW + _CPW * _CHUNK, _TAIL)],
                         w2.at[pl.ds(0, _TAIL)], sm2)
        pltpu.make_async_copy(h_hbm.at[ij_t], r2.at[pl.ds(0, _TAIL)],
                              sm2).wait()
        pltpu.make_async_copy(
            wc_hbm.at[pl.ds(wid * _EPW + _CPW * _CHUNK, _TAIL)],
            w2.at[pl.ds(0, _TAIL)], sm2).wait()
        compute(2, n=_TAIL)
        wait_scatter(cb, 1)
        pltpu.async_copy(r2.at[pl.ds(0, _TAIL)], acc_sh.at[ii_t], ss2,
                         add=True)
        pltpu.make_async_copy(r2.at[pl.ds(0, _TAIL)], acc_sh.at[ii_t],
                              ss2).wait()

        plsc.subcore_barrier()
        pltpu.sync_copy(acc_sh.at[pl.ds(sid * _RPT, _RPT)],
                        out_hbm.at[pl.ds(cid * _NP + sid * _RPT, _RPT)])

    return k


_sc_aggregate_a = _make_sc_aggregate(0)
_sc_aggregate_b = _make_sc_aggregate(_EH)


def _out_body(pa_ref, pb_ref, w1_ref, b1_ref, w2_ref, b2_ref, o_ref):
    agg = (pa_ref[0, :_N, :] + pa_ref[1, :_N, :]
           + pb_ref[0, :_N, :] + pb_ref[1, :_N, :])
    t = lax.dot_general(agg, w1_ref[...], (((1,), (1,)), ((), ())),
                        preferred_element_type=jnp.float32)
    t = _ssp(t + b1_ref[...])
    o = lax.dot_general(t, w2_ref[...], (((1,), (1,)), ((), ())),
                        preferred_element_type=jnp.float32)
    o_ref[...] = o + b2_ref[...]


def _out_mlp(pa, pb, Wo1, bo1, Wo2, bo2):
    return pl.pallas_call(
        _out_body,
        out_shape=jax.ShapeDtypeStruct((_N, _D), jnp.float32),
    )(pa, pb, Wo1, bo1, Wo2, bo2)


def kernel(x, pairlist, f_ij, f_ij_cutoff,
           W_in, Wf1, bf1, Wf2, bf2, Wo1, bo1, Wo2, bo2):
    h = _compute_h(x, W_in)
    b1 = bf1.reshape(1, _F)
    b2 = bf2.reshape(1, _F)
    f2d_a = f_ij[:_EH].reshape(_EH, _R)
    f2d_b = f_ij[_EH:].reshape(_EH, _R)
    wc_a = _compute_wc_a(f2d_a, f_ij_cutoff, Wf1, b1, Wf2, b2)
    wc_b = _compute_wc_b(f2d_b, f_ij_cutoff, Wf1, b1, Wf2, b2)
    zeros = jnp.zeros((_RPT, _D), jnp.float32)
    ii = pairlist[0]
    ij = pairlist[1]
    pa = _sc_aggregate_a(h, wc_a, ii, ij, zeros)
    pb = _sc_aggregate_b(h, wc_b, ii, ij, zeros)
    out = _out_mlp(pa.reshape(_NC, _NP, _D), pb.reshape(_NC, _NP, _D),
                   Wo1, bo1.reshape(1, _D), Wo2, bo2.reshape(1, _D))
    return out


# speedup vs baseline: 1.5831x; 1.1106x over previous
import functools

import jax
import jax.numpy as jnp
from jax import lax
from jax.experimental import pallas as pl
from jax.experimental.pallas import tpu as pltpu
from jax.experimental.pallas import tpu_sc as plsc

_N = 10000
_E = 320000
_D = 128
_F = 128
_R = 16

_EH = _E // 2
_NC = 2
_NS = 16
_NW = _NC * _NS
_EPW = _EH // _NW
_CHUNK = 56
_CPW = _EPW // _CHUNK
_TAIL = _EPW - _CPW * _CHUNK
_NT = (_CPW - 2) // 3
_NP = 10240
_RPT = _NP // _NS

_LOG2 = 0.6931471805599453
_LOG2E = 1.4426950408889634


def _ssp(v):
    return jax.nn.softplus(v) - _LOG2


def _ssp_fast(u):
    t = jnp.log2(1.0 + jnp.exp2(u * _LOG2E)) * _LOG2 - _LOG2
    return jnp.where(u > 60.0, u - _LOG2, t)


def _h_body(x_ref, w_ref, o_ref):
    o_ref[...] = lax.dot_general(
        x_ref[...], w_ref[...], (((1,), (1,)), ((), ())),
        preferred_element_type=jnp.float32)


def _compute_h(x, W_in):
    return pl.pallas_call(
        _h_body,
        out_shape=jax.ShapeDtypeStruct((_N, _D), jnp.float32),
    )(x, W_in)


_BE = 4000


def _wc_body(f_ref, c_ref, w1_ref, b1_ref, w2_ref, b2_ref, o_ref):
    t = lax.dot_general(f_ref[...], w1_ref[...], (((1,), (1,)), ((), ())),
                        preferred_element_type=jnp.float32)
    t = _ssp_fast(t + b1_ref[...])
    w = lax.dot_general(t.astype(jnp.bfloat16),
                        w2_ref[...].astype(jnp.bfloat16),
                        (((1,), (1,)), ((), ())),
                        preferred_element_type=jnp.float32)
    o_ref[...] = (w + b2_ref[...]) * c_ref[...]


def _make_wc(off_blocks):
    return pl.pallas_call(
        _wc_body,
        grid=(_EH // _BE,),
        in_specs=[
            pl.BlockSpec((_BE, _R), lambda i: (i + off_blocks, 0)),
            pl.BlockSpec((_BE, 1), lambda i: (i + off_blocks, 0)),
            pl.BlockSpec((_F, _R), lambda i: (0, 0)),
            pl.BlockSpec((1, _F), lambda i: (0, 0)),
            pl.BlockSpec((_F, _F), lambda i: (0, 0)),
            pl.BlockSpec((1, _F), lambda i: (0, 0)),
        ],
        out_specs=pl.BlockSpec((_BE, _F), lambda i: (i, 0)),
        out_shape=jax.ShapeDtypeStruct((_EH, _F), jnp.float32),
    )


_compute_wc_a = _make_wc(0)
_compute_wc_b = _make_wc(_EH // _BE)


def _make_sc_aggregate(eoff):
    mesh = plsc.VectorSubcoreMesh(core_axis_name="c", subcore_axis_name="s")

    @functools.partial(
        pl.kernel,
        out_type=jax.ShapeDtypeStruct((_NC * _NP, _D), jnp.float32),
        mesh=mesh,
        scratch_types=[
            pltpu.VMEM((3, _CHUNK), jnp.int32),
            pltpu.VMEM((3, _CHUNK), jnp.int32),
            pltpu.VMEM((_TAIL,), jnp.int32),
            pltpu.VMEM((_TAIL,), jnp.int32),
            pltpu.VMEM((_CHUNK, _D), jnp.float32),
            pltpu.VMEM((_CHUNK, _D), jnp.float32),
            pltpu.VMEM((_CHUNK, _D), jnp.float32),
            pltpu.VMEM((_CHUNK, _D), jnp.float32),
            pltpu.VMEM((_CHUNK, _D), jnp.float32),
            pltpu.VMEM((_CHUNK, _D), jnp.float32),
            pltpu.VMEM_SHARED((_NP, _D), jnp.float32),
            pltpu.SemaphoreType.DMA,
            pltpu.SemaphoreType.DMA,
            pltpu.SemaphoreType.DMA,
            pltpu.SemaphoreType.DMA,
            pltpu.SemaphoreType.DMA,
            pltpu.SemaphoreType.DMA,
            pltpu.SemaphoreType.DMA,
            pltpu.SemaphoreType.DMA,
            pltpu.SemaphoreType.DMA,
            pltpu.SemaphoreType.DMA,
        ],
    )
    def k(h_hbm, wc_hbm, ii_hbm, ij_hbm, z_hbm, out_hbm,
          ii_v, ij_v, ii_t, ij_t, r0, r1, r2, w0, w1, w2, acc_sh,
          sem_z, si0, si1, si2, sm0, sm1, sm2, ss0, ss1, ss2):
        cid = lax.axis_index("c")
        sid = lax.axis_index("s")
        wid = sid * _NC + cid
        rbufs = (r0, r1, r2)
        wbufs = (w0, w1, w2)
        isems = (si0, si1, si2)
        msems = (sm0, sm1, sm2)
        ssems = (ss0, ss1, ss2)

        pltpu.async_copy(z_hbm, acc_sh.at[pl.ds(sid * _RPT, _RPT)], sem_z)

        def lbase(c):
            return wid * _EPW + c * _CHUNK

        def issue_idx(c, b):
            pltpu.async_copy(ii_hbm.at[pl.ds(eoff + lbase(c), _CHUNK)],
                             ii_v.at[b], isems[b])
            pltpu.async_copy(ij_hbm.at[pl.ds(eoff + lbase(c), _CHUNK)],
                             ij_v.at[b], isems[b])

        def wait_idx(c, b):
            pltpu.make_async_copy(ii_hbm.at[pl.ds(eoff + lbase(c), _CHUNK)],
                                  ii_v.at[b], isems[b]).wait()
            pltpu.make_async_copy(ij_hbm.at[pl.ds(eoff + lbase(c), _CHUNK)],
                                  ij_v.at[b], isems[b]).wait()

        def issue_main(c, b):
            pltpu.async_copy(h_hbm.at[ij_v.at[b]], rbufs[b], msems[b])
            pltpu.async_copy(wc_hbm.at[pl.ds(lbase(c), _CHUNK)],
                             wbufs[b], msems[b])

        def wait_main(c, b):
            pltpu.make_async_copy(h_hbm.at[ij_v.at[b]], rbufs[b],
                                  msems[b]).wait()
            pltpu.make_async_copy(wc_hbm.at[pl.ds(lbase(c), _CHUNK)],
                                  wbufs[b], msems[b]).wait()

        def issue_scatter(c, b):
            pltpu.async_copy(rbufs[b], acc_sh.at[ii_v.at[b]], ssems[b],
                             add=True)

        def wait_scatter(c, b):
            pltpu.make_async_copy(rbufs[b], acc_sh.at[ii_v.at[b]],
                                  ssems[b]).wait()

        def compute(b, n=_CHUNK):
            rb, wb = rbufs[b], wbufs[b]

            @pl.loop(0, n)
            def _(e):
                for j in range(0, _D, 16):
                    slc = (pl.ds(e, 1), pl.ds(j, 16))
                    rb.at[slc][...] = rb.at[slc][...] * wb.at[slc][...]

        pltpu.make_async_copy(z_hbm, acc_sh.at[pl.ds(sid * _RPT, _RPT)],
                              sem_z).wait()
        plsc.subcore_barrier()

        issue_idx(0, 0)
        issue_idx(1, 1)
        wait_idx(0, 0)
        issue_main(0, 0)

        @pl.loop(0, _NT)
        def _(t):
            for j in range(3):
                c = t * 3 + j
                b = j
                bn = (j + 2) % 3

                wait_idx(c + 1, (b + 1) % 3)
                issue_main(c + 1, (b + 1) % 3)

                wait_main(c, b)
                compute(b)

                if j == 0:
                    @pl.when(t > 0)
                    def _():
                        wait_scatter(c - 1, bn)
                        issue_idx(c + 2, bn)

                    @pl.when(t == 0)
                    def _():
                        issue_idx(c + 2, bn)
                else:
                    wait_scatter(c - 1, bn)
                    if j == 2:
                        @pl.when(t < _NT - 1)
                        def _():
                            issue_idx(c + 2, bn)
                    else:
                        issue_idx(c + 2, bn)

                issue_scatter(c, b)

        ca = 3 * _NT
        cb = ca + 1
        issue_idx(cb, 1)
        wait_idx(cb, 1)
        issue_main(cb, 1)
        wait_main(ca, 0)
        compute(0)
        wait_scatter(ca - 1, 2)
        issue_scatter(ca, 0)

        tbase = eoff + wid * _EPW + _CPW * _CHUNK
        pltpu.async_copy(ii_hbm.at[pl.ds(tbase, _TAIL)], ii_t, si2)
        pltpu.async_copy(ij_hbm.at[pl.ds(tbase, _TAIL)], ij_t, si2)
        wait_main(cb, 1)
        compute(1)
        wait_scatter(ca, 0)
        issue_scatter(cb, 1)

        pltpu.make_async_copy(ii_hbm.at[pl.ds(tbase, _TAIL)], ii_t,
                              si2).wait()
        pltpu.make_async_copy(ij_hbm.at[pl.ds(tbase, _TAIL)], ij_t,
                              si2).wait()
        pltpu.async_copy(h_hbm.at[ij_t], r2.at[pl.ds(0, _TAIL)], sm2)
        pltpu.async_copy(wc_hbm.at[pl.ds(wid * _EPW + _CPW * _CHUNK, _TAIL)],
                         w2.at[pl.ds(0, _TAIL)], sm2)
        pltpu.make_async_copy(h_hbm.at[ij_t], r2.at[pl.ds(0, _TAIL)],
                              sm2).wait()
        pltpu.make_async_copy(
            wc_hbm.at[pl.ds(wid * _EPW + _CPW * _CHUNK, _TAIL)],
            w2.at[pl.ds(0, _TAIL)], sm2).wait()
        compute(2, n=_TAIL)
        wait_scatter(cb, 1)
        pltpu.async_copy(r2.at[pl.ds(0, _TAIL)], acc_sh.at[ii_t], ss2,
                         add=True)
        pltpu.make_async_copy(r2.at[pl.ds(0, _TAIL)], acc_sh.at[ii_t],
                              ss2).wait()

        plsc.subcore_barrier()
        pltpu.sync_copy(acc_sh.at[pl.ds(sid * _RPT, _RPT)],
                        out_hbm.at[pl.ds(cid * _NP + sid * _RPT, _RPT)])

    return k


_sc_aggregate_a = _make_sc_aggregate(0)
_sc_aggregate_b = _make_sc_aggregate(_EH)


def _out_body(pa_ref, pb_ref, w1_ref, b1_ref, w2_ref, b2_ref, o_ref):
    agg = (pa_ref[0, :_N, :] + pa_ref[1, :_N, :]
           + pb_ref[0, :_N, :] + pb_ref[1, :_N, :])
    t = lax.dot_general(agg, w1_ref[...], (((1,), (1,)), ((), ())),
                        preferred_element_type=jnp.float32)
    t = _ssp(t + b1_ref[...])
    o = lax.dot_general(t, w2_ref[...], (((1,), (1,)), ((), ())),
                        preferred_element_type=jnp.float32)
    o_ref[...] = o + b2_ref[...]


def _out_mlp(pa, pb, Wo1, bo1, Wo2, bo2):
    return pl.pallas_call(
        _out_body,
        out_shape=jax.ShapeDtypeStruct((_N, _D), jnp.float32),
    )(pa, pb, Wo1, bo1, Wo2, bo2)


def kernel(x, pairlist, f_ij, f_ij_cutoff,
           W_in, Wf1, bf1, Wf2, bf2, Wo1, bo1, Wo2, bo2):
    h = _compute_h(x, W_in)
    b1 = bf1.reshape(1, _F)
    b2 = bf2.reshape(1, _F)
    f2d = f_ij.reshape(_E, _R)
    wc_a = _compute_wc_a(f2d, f_ij_cutoff, Wf1, b1, Wf2, b2)
    wc_b = _compute_wc_b(f2d, f_ij_cutoff, Wf1, b1, Wf2, b2)
    zeros = jnp.zeros((_RPT, _D), jnp.float32)
    ii = pairlist[0]
    ij = pairlist[1]
    pa = _sc_aggregate_a(h, wc_a, ii, ij, zeros)
    pb = _sc_aggregate_b(h, wc_b, ii, ij, zeros)
    out = _out_mlp(pa.reshape(_NC, _NP, _D), pb.reshape(_NC, _NP, _D),
                   Wo1, bo1.reshape(1, _D), Wo2, bo2.reshape(1, _D))
    return out


# speedup vs baseline: 1.5919x; 1.0056x over previous
import functools

import jax
import jax.numpy as jnp
from jax import lax
from jax.experimental import pallas as pl
from jax.experimental.pallas import tpu as pltpu
from jax.experimental.pallas import tpu_sc as plsc

_N = 10000
_E = 320000
_D = 128
_F = 128
_R = 16

_EH = _E // 2
_NC = 2
_NS = 16
_NW = _NC * _NS
_EPW = _EH // _NW
_CHUNK = 56
_CPW = _EPW // _CHUNK
_TAIL = _EPW - _CPW * _CHUNK
_NT = (_CPW - 2) // 3
_NP = 10240
_RPT = _NP // _NS

_LOG2 = 0.6931471805599453
_LOG2E = 1.4426950408889634


def _ssp(v):
    return jax.nn.softplus(v) - _LOG2


def _ssp_fast(u):
    t = jnp.log2(1.0 + jnp.exp2(u * _LOG2E)) * _LOG2 - _LOG2
    return jnp.where(u > 60.0, u - _LOG2, t)


def _h_body(x_ref, w_ref, o_ref):
    o_ref[...] = lax.dot_general(
        x_ref[...], w_ref[...], (((1,), (1,)), ((), ())),
        preferred_element_type=jnp.float32)


def _compute_h(x, W_in):
    return pl.pallas_call(
        _h_body,
        out_shape=jax.ShapeDtypeStruct((_N, _D), jnp.float32),
    )(x, W_in)


_BE = 8000


def _wc_body(f_ref, c_ref, w1_ref, b1_ref, w2_ref, b2_ref, o_ref):
    t = lax.dot_general(f_ref[...], w1_ref[...], (((1,), (1,)), ((), ())),
                        preferred_element_type=jnp.float32)
    t = _ssp_fast(t + b1_ref[...])
    w = lax.dot_general(t.astype(jnp.bfloat16),
                        w2_ref[...].astype(jnp.bfloat16),
                        (((1,), (1,)), ((), ())),
                        preferred_element_type=jnp.float32)
    o_ref[...] = (w + b2_ref[...]) * c_ref[...]


def _make_wc(off_blocks):
    return pl.pallas_call(
        _wc_body,
        grid=(_EH // _BE,),
        in_specs=[
            pl.BlockSpec((_BE, _R), lambda i: (i + off_blocks, 0)),
            pl.BlockSpec((_BE, 1), lambda i: (i + off_blocks, 0)),
            pl.BlockSpec((_F, _R), lambda i: (0, 0)),
            pl.BlockSpec((1, _F), lambda i: (0, 0)),
            pl.BlockSpec((_F, _F), lambda i: (0, 0)),
            pl.BlockSpec((1, _F), lambda i: (0, 0)),
        ],
        out_specs=pl.BlockSpec((_BE, _F), lambda i: (i, 0)),
        out_shape=jax.ShapeDtypeStruct((_EH, _F), jnp.float32),
    )


_compute_wc_a = _make_wc(0)
_compute_wc_b = _make_wc(_EH // _BE)


def _make_sc_aggregate(eoff):
    mesh = plsc.VectorSubcoreMesh(core_axis_name="c", subcore_axis_name="s")

    @functools.partial(
        pl.kernel,
        out_type=jax.ShapeDtypeStruct((_NC * _NP, _D), jnp.float32),
        mesh=mesh,
        scratch_types=[
            pltpu.VMEM((3, _CHUNK), jnp.int32),
            pltpu.VMEM((3, _CHUNK), jnp.int32),
            pltpu.VMEM((_TAIL,), jnp.int32),
            pltpu.VMEM((_TAIL,), jnp.int32),
            pltpu.VMEM((_CHUNK, _D), jnp.float32),
            pltpu.VMEM((_CHUNK, _D), jnp.float32),
            pltpu.VMEM((_CHUNK, _D), jnp.float32),
            pltpu.VMEM((_CHUNK, _D), jnp.float32),
            pltpu.VMEM((_CHUNK, _D), jnp.float32),
            pltpu.VMEM((_CHUNK, _D), jnp.float32),
            pltpu.VMEM_SHARED((_NP, _D), jnp.float32),
            pltpu.SemaphoreType.DMA,
            pltpu.SemaphoreType.DMA,
            pltpu.SemaphoreType.DMA,
            pltpu.SemaphoreType.DMA,
            pltpu.SemaphoreType.DMA,
            pltpu.SemaphoreType.DMA,
            pltpu.SemaphoreType.DMA,
            pltpu.SemaphoreType.DMA,
            pltpu.SemaphoreType.DMA,
            pltpu.SemaphoreType.DMA,
        ],
    )
    def k(h_hbm, wc_hbm, ii_hbm, ij_hbm, z_hbm, out_hbm,
          ii_v, ij_v, ii_t, ij_t, r0, r1, r2, w0, w1, w2, acc_sh,
          sem_z, si0, si1, si2, sm0, sm1, sm2, ss0, ss1, ss2):
        cid = lax.axis_index("c")
        sid = lax.axis_index("s")
        wid = sid * _NC + cid
        rbufs = (r0, r1, r2)
        wbufs = (w0, w1, w2)
        isems = (si0, si1, si2)
        msems = (sm0, sm1, sm2)
        ssems = (ss0, ss1, ss2)

        pltpu.async_copy(z_hbm, acc_sh.at[pl.ds(sid * _RPT, _RPT)], sem_z)

        def lbase(c):
            return wid * _EPW + c * _CHUNK

        def issue_idx(c, b):
            pltpu.async_copy(ii_hbm.at[pl.ds(eoff + lbase(c), _CHUNK)],
                             ii_v.at[b], isems[b])
            pltpu.async_copy(ij_hbm.at[pl.ds(eoff + lbase(c), _CHUNK)],
                             ij_v.at[b], isems[b])

        def wait_idx(c, b):
            pltpu.make_async_copy(ii_hbm.at[pl.ds(eoff + lbase(c), _CHUNK)],
                                  ii_v.at[b], isems[b]).wait()
            pltpu.make_async_copy(ij_hbm.at[pl.ds(eoff + lbase(c), _CHUNK)],
                                  ij_v.at[b], isems[b]).wait()

        def issue_main(c, b):
            pltpu.async_copy(h_hbm.at[ij_v.at[b]], rbufs[b], msems[b])
            pltpu.async_copy(wc_hbm.at[pl.ds(lbase(c), _CHUNK)],
                             wbufs[b], msems[b])

        def wait_main(c, b):
            pltpu.make_async_copy(h_hbm.at[ij_v.at[b]], rbufs[b],
                                  msems[b]).wait()
            pltpu.make_async_copy(wc_hbm.at[pl.ds(lbase(c), _CHUNK)],
                                  wbufs[b], msems[b]).wait()

        def issue_scatter(c, b):
            pltpu.async_copy(rbufs[b], acc_sh.at[ii_v.at[b]], ssems[b],
                             add=True)

        def wait_scatter(c, b):
            pltpu.make_async_copy(rbufs[b], acc_sh.at[ii_v.at[b]],
                                  ssems[b]).wait()

        def compute(b, n=_CHUNK):
            rb, wb = rbufs[b], wbufs[b]

            @pl.loop(0, n)
            def _(e):
                for j in range(0, _D, 16):
                    slc = (pl.ds(e, 1), pl.ds(j, 16))
                    rb.at[slc][...] = rb.at[slc][...] * wb.at[slc][...]

        pltpu.make_async_copy(z_hbm, acc_sh.at[pl.ds(sid * _RPT, _RPT)],
                              sem_z).wait()
        plsc.subcore_barrier()

        issue_idx(0, 0)
        issue_idx(1, 1)
        wait_idx(0, 0)
        issue_main(0, 0)

        @pl.loop(0, _NT)
        def _(t):
            for j in range(3):
                c = t * 3 + j
                b = j
                bn = (j + 2) % 3

                wait_idx(c + 1, (b + 1) % 3)
                issue_main(c + 1, (b + 1) % 3)

                wait_main(c, b)
                compute(b)

                if j == 0:
                    @pl.when(t > 0)
                    def _():
                        wait_scatter(c - 1, bn)
                        issue_idx(c + 2, bn)

                    @pl.when(t == 0)
                    def _():
                        issue_idx(c + 2, bn)
                else:
                    wait_scatter(c - 1, bn)
                    if j == 2:
                        @pl.when(t < _NT - 1)
                        def _():
                            issue_idx(c + 2, bn)
                    else:
                        issue_idx(c + 2, bn)

                issue_scatter(c, b)

        ca = 3 * _NT
        cb = ca + 1
        issue_idx(cb, 1)
        wait_idx(cb, 1)
        issue_main(cb, 1)
        wait_main(ca, 0)
        compute(0)
        wait_scatter(ca - 1, 2)
        issue_scatter(ca, 0)

        tbase = eoff + wid * _EPW + _CPW * _CHUNK
        pltpu.async_copy(ii_hbm.at[pl.ds(tbase, _TAIL)], ii_t, si2)
        pltpu.async_copy(ij_hbm.at[pl.ds(tbase, _TAIL)], ij_t, si2)
        wait_main(cb, 1)
        compute(1)
        wait_scatter(ca, 0)
        issue_scatter(cb, 1)

        pltpu.make_async_copy(ii_hbm.at[pl.ds(tbase, _TAIL)], ii_t,
                              si2).wait()
        pltpu.make_async_copy(ij_hbm.at[pl.ds(tbase, _TAIL)], ij_t,
                              si2).wait()
        pltpu.async_copy(h_hbm.at[ij_t], r2.at[pl.ds(0, _TAIL)], sm2)
        pltpu.async_copy(wc_hbm.at[pl.ds(wid * _EPW + _CPW * _CHUNK, _TAIL)],
                         w2.at[pl.ds(0, _TAIL)], sm2)
        pltpu.make_async_copy(h_hbm.at[ij_t], r2.at[pl.ds(0, _TAIL)],
                              sm2).wait()
        pltpu.make_async_copy(
            wc_hbm.at[pl.ds(wid * _EPW + _CPW * _CHUNK, _TAIL)],
            w2.at[pl.ds(0, _TAIL)], sm2).wait()
        compute(2, n=_TAIL)
        wait_scatter(cb, 1)
        pltpu.async_copy(r2.at[pl.ds(0, _TAIL)], acc_sh.at[ii_t], ss2,
                         add=True)
        pltpu.make_async_copy(r2.at[pl.ds(0, _TAIL)], acc_sh.at[ii_t],
                              ss2).wait()

        plsc.subcore_barrier()
        pltpu.sync_copy(acc_sh.at[pl.ds(sid * _RPT, _RPT)],
                        out_hbm.at[pl.ds(cid * _NP + sid * _RPT, _RPT)])

    return k


_sc_aggregate_a = _make_sc_aggregate(0)
_sc_aggregate_b = _make_sc_aggregate(_EH)


def _out_body(pa_ref, pb_ref, w1_ref, b1_ref, w2_ref, b2_ref, o_ref):
    agg = (pa_ref[0, :_N, :] + pa_ref[1, :_N, :]
           + pb_ref[0, :_N, :] + pb_ref[1, :_N, :])
    t = lax.dot_general(agg, w1_ref[...], (((1,), (1,)), ((), ())),
                        preferred_element_type=jnp.float32)
    t = _ssp(t + b1_ref[...])
    o = lax.dot_general(t, w2_ref[...], (((1,), (1,)), ((), ())),
                        preferred_element_type=jnp.float32)
    o_ref[...] = o + b2_ref[...]


def _out_mlp(pa, pb, Wo1, bo1, Wo2, bo2):
    return pl.pallas_call(
        _out_body,
        out_shape=jax.ShapeDtypeStruct((_N, _D), jnp.float32),
    )(pa, pb, Wo1, bo1, Wo2, bo2)


def kernel(x, pairlist, f_ij, f_ij_cutoff,
           W_in, Wf1, bf1, Wf2, bf2, Wo1, bo1, Wo2, bo2):
    h = _compute_h(x, W_in)
    b1 = bf1.reshape(1, _F)
    b2 = bf2.reshape(1, _F)
    f2d = f_ij.reshape(_E, _R)
    wc_a = _compute_wc_a(f2d, f_ij_cutoff, Wf1, b1, Wf2, b2)
    wc_b = _compute_wc_b(f2d, f_ij_cutoff, Wf1, b1, Wf2, b2)
    zeros = jnp.zeros((_RPT, _D), jnp.float32)
    ii = pairlist[0]
    ij = pairlist[1]
    pa = _sc_aggregate_a(h, wc_a, ii, ij, zeros)
    pb = _sc_aggregate_b(h, wc_b, ii, ij, zeros)
    out = _out_mlp(pa.reshape(_NC, _NP, _D), pb.reshape(_NC, _NP, _D),
                   Wo1, bo1.reshape(1, _D), Wo2, bo2.reshape(1, _D))
    return out
